# baseline (speedup 1.0000x reference)
import functools

import jax
import jax.numpy as jnp
from jax.experimental import pallas as pl

N = 10000
D = 128

_SKIP_TO = [[2, 4], [3], [4], [], []]
_ACTS = ['relu', 'elu', 'sigmoid', 'relu', 'elu']

_BI = 400
_BR = 2000


def _apply_act(h, act):
    if act == 'relu':
        return jnp.maximum(h, 0.0)
    if act == 'sigmoid':
        return jax.nn.sigmoid(h)
    return jnp.where(h > 0, h, jnp.exp(jnp.minimum(h, 0.0)) - 1.0)


def _merge_matmul_kernel(*refs):
    *x_refs, w_ref, o_ref = refs
    s = x_refs[0][...]
    for r in x_refs[1:]:
        s = s + r[...]
    s = jnp.maximum(s, 0.0)
    o_ref[...] = jnp.dot(s, w_ref[...], preferred_element_type=jnp.float32)


def _merge_matmul(parts, w):
    return pl.pallas_call(
        _merge_matmul_kernel,
        grid=(N // _BR,),
        in_specs=[pl.BlockSpec((_BR, D), lambda i: (i, 0))] * len(parts)
        + [pl.BlockSpec((D, D), lambda i: (0, 0))],
        out_specs=pl.BlockSpec((_BR, D), lambda i: (i, 0)),
        out_shape=jax.ShapeDtypeStruct((N, D), jnp.float32),
    )(*parts, w)


def _adj_layer_kernel(adj_ref, y_ref, b_ref, o_ref, *, act):
    h = jnp.dot(adj_ref[...], y_ref[...], preferred_element_type=jnp.float32)
    h = h + b_ref[...]
    o_ref[...] = _apply_act(h, act)


def _adj_layer(adj, y, b2d, act):
    return pl.pallas_call(
        functools.partial(_adj_layer_kernel, act=act),
        grid=(N // _BI,),
        in_specs=[
            pl.BlockSpec((_BI, N), lambda i: (i, 0)),
            pl.BlockSpec((N, D), lambda i: (0, 0)),
            pl.BlockSpec((1, D), lambda i: (0, 0)),
        ],
        out_specs=pl.BlockSpec((_BI, D), lambda i: (i, 0)),
        out_shape=jax.ShapeDtypeStruct((N, D), jnp.float32),
    )(adj, y, b2d)


def kernel(x, adj, W1, b1, W2, b2, W3, b3, W4, b4, W5, b5):
    Ws = [W1, W2, W3, W4, W5]
    bs = [b.reshape(1, D) for b in (b1, b2, b3, b4, b5)]
    outs = []
    cur = x
    for i in range(5):
        parts = [cur] + [outs[j] for j in range(i) if i in _SKIP_TO[j]]
        y = _merge_matmul(parts, Ws[i])
        cur = _adj_layer(adj, y, bs[i], _ACTS[i])
        outs.append(cur)
    return cur


# speedup vs baseline: 1.2461x; 1.2461x over previous
import functools

import jax
import jax.numpy as jnp
from jax.experimental import pallas as pl

N = 10000
D = 128

_SKIP_TO = [[2, 4], [3], [4], [], []]
_ACTS = ['relu', 'elu', 'sigmoid', 'relu', 'elu']

_BI = 400
_BR = 2000


def _apply_act(h, act):
    if act == 'relu':
        return jnp.maximum(h, 0.0)
    if act == 'sigmoid':
        return jax.nn.sigmoid(h)
    return jnp.where(h > 0, h, jnp.exp(jnp.minimum(h, 0.0)) - 1.0)


def _merge_matmul_kernel(*refs):
    *x_refs, w_ref, o_ref = refs
    s = x_refs[0][...]
    for r in x_refs[1:]:
        s = s + r[...]
    s = jnp.maximum(s, 0.0)
    o_ref[...] = jnp.dot(
        s, w_ref[...], preferred_element_type=jnp.float32
    ).astype(jnp.bfloat16)


def _merge_matmul(parts, w):
    return pl.pallas_call(
        _merge_matmul_kernel,
        grid=(N // _BR,),
        in_specs=[pl.BlockSpec((_BR, D), lambda i: (i, 0))] * len(parts)
        + [pl.BlockSpec((D, D), lambda i: (0, 0))],
        out_specs=pl.BlockSpec((_BR, D), lambda i: (i, 0)),
        out_shape=jax.ShapeDtypeStruct((N, D), jnp.bfloat16),
    )(*parts, w)


def _layer1_kernel(adj_ref, y_ref, b_ref, o_ref, adjb_ref, *, act):
    ab = adj_ref[...].astype(jnp.bfloat16)
    adjb_ref[...] = ab
    h = jnp.dot(ab, y_ref[...], preferred_element_type=jnp.float32)
    o_ref[...] = _apply_act(h + b_ref[...], act)


def _layer1(adj, y, b2d, act):
    return pl.pallas_call(
        functools.partial(_layer1_kernel, act=act),
        grid=(N // _BI,),
        in_specs=[
            pl.BlockSpec((_BI, N), lambda i: (i, 0)),
            pl.BlockSpec((N, D), lambda i: (0, 0)),
            pl.BlockSpec((1, D), lambda i: (0, 0)),
        ],
        out_specs=[
            pl.BlockSpec((_BI, D), lambda i: (i, 0)),
            pl.BlockSpec((_BI, N), lambda i: (i, 0)),
        ],
        out_shape=[
            jax.ShapeDtypeStruct((N, D), jnp.float32),
            jax.ShapeDtypeStruct((N, N), jnp.bfloat16),
        ],
    )(adj, y, b2d)


def _adj_layer_kernel(adj_ref, y_ref, b_ref, o_ref, *, act):
    h = jnp.dot(adj_ref[...], y_ref[...], preferred_element_type=jnp.float32)
    o_ref[...] = _apply_act(h + b_ref[...], act)


def _adj_layer(adj_bf16, y, b2d, act):
    return pl.pallas_call(
        functools.partial(_adj_layer_kernel, act=act),
        grid=(N // _BI,),
        in_specs=[
            pl.BlockSpec((_BI, N), lambda i: (i, 0)),
            pl.BlockSpec((N, D), lambda i: (0, 0)),
            pl.BlockSpec((1, D), lambda i: (0, 0)),
        ],
        out_specs=pl.BlockSpec((_BI, D), lambda i: (i, 0)),
        out_shape=jax.ShapeDtypeStruct((N, D), jnp.float32),
    )(adj_bf16, y, b2d)


def kernel(x, adj, W1, b1, W2, b2, W3, b3, W4, b4, W5, b5):
    Ws = [W1, W2, W3, W4, W5]
    bs = [b.reshape(1, D) for b in (b1, b2, b3, b4, b5)]
    outs = []
    cur = x
    adj_bf16 = None
    for i in range(5):
        parts = [cur] + [outs[j] for j in range(i) if i in _SKIP_TO[j]]
        y = _merge_matmul(parts, Ws[i])
        if i == 0:
            cur, adj_bf16 = _layer1(adj, y, bs[i], _ACTS[i])
        else:
            cur = _adj_layer(adj_bf16, y, bs[i], _ACTS[i])
        outs.append(cur)
    return cur
